# Initial kernel scaffold; baseline (speedup 1.0000x reference)
import jax
import jax.numpy as jnp
from jax.experimental import pallas as pl


def kernel(input_ids, pos_embedding):
    raise NotImplementedError("write your pallas kernel here")


# speedup vs baseline: 5.0431x; 5.0431x over previous
import functools

import jax
import jax.numpy as jnp
from jax import lax
from jax.experimental import pallas as pl
from jax.experimental.pallas import tpu as pltpu
from jax.experimental.pallas import tpu_sc as plsc

_SEQ = 200
_DIM = 64
_BATCH = 4096
_ROW = _SEQ * _DIM
_REP = 8


@jax.jit
def _pos_broadcast(pos_embedding):
    info = plsc.get_sparse_core_info()
    nw = info.num_cores * info.num_subcores
    per_w = _BATCH // nw
    n_dma = per_w // _REP

    mesh = plsc.VectorSubcoreMesh(core_axis_name="c", subcore_axis_name="s")

    @functools.partial(
        pl.kernel,
        mesh=mesh,
        out_type=jax.ShapeDtypeStruct((_BATCH * _ROW,), jnp.float32),
        scratch_types=[
            pltpu.VMEM((_REP * _ROW,), jnp.float32),
            pltpu.SemaphoreType.DMA,
        ],
    )
    def k(table_hbm, out_hbm, rep_v, sem):
        for r in range(_REP):
            pltpu.sync_copy(table_hbm, rep_v.at[pl.ds(r * _ROW, _ROW)])
        wid = lax.axis_index("s") * info.num_cores + lax.axis_index("c")
        base = wid * per_w * _ROW
        copies = [
            pltpu.async_copy(
                rep_v, out_hbm.at[pl.ds(base + i * _REP * _ROW, _REP * _ROW)], sem
            )
            for i in range(n_dma)
        ]
        for c in copies:
            c.wait()

    flat = k(pos_embedding.reshape(_ROW))
    return flat.reshape(_BATCH, _SEQ, _DIM)


def kernel(input_ids, pos_embedding):
    del input_ids
    return _pos_broadcast(pos_embedding)
